# baseline (speedup 1.0000x reference)
import jax
import jax.numpy as jnp
from jax.experimental import pallas as pl
from jax.experimental.pallas import tpu as pltpu

N_TOKENS = 9216
QUANT_DIM = 1024
OUTPUT_DIM = 256

_TB = 2304


def _body(x_ref, w_hbm, out_ref, w_vmem, sem):
    @pl.when(pl.program_id(0) == 0)
    def _():
        cp = pltpu.make_async_copy(w_hbm, w_vmem, sem)
        cp.start()
        cp.wait()

    xb = x_ref[...]
    m = jnp.max(xb, axis=-1, keepdims=True)
    iota = jax.lax.broadcasted_iota(jnp.int32, (_TB, QUANT_DIM), 1)
    idx = jnp.min(jnp.where(xb == m, iota, QUANT_DIM), axis=-1)
    oh = (iota == idx[:, None]).astype(jnp.bfloat16)
    dims = (((1,), (1,)), ((), ()))
    out_ref[...] = jax.lax.dot_general(
        oh, w_vmem[...].astype(jnp.bfloat16), dims,
        preferred_element_type=jnp.float32,
    )


def kernel(x, W):
    grid = N_TOKENS // _TB
    return pl.pallas_call(
        _body,
        grid=(grid,),
        in_specs=[
            pl.BlockSpec((_TB, QUANT_DIM), lambda i: (i, 0)),
            pl.BlockSpec(memory_space=pltpu.MemorySpace.HBM),
        ],
        out_specs=pl.BlockSpec((_TB, OUTPUT_DIM), lambda i: (i, 0)),
        out_shape=jax.ShapeDtypeStruct((N_TOKENS, OUTPUT_DIM), jnp.float32),
        scratch_shapes=[
            pltpu.VMEM((OUTPUT_DIM, QUANT_DIM), jnp.float32),
            pltpu.SemaphoreType.DMA,
        ],
    )(x, W)


# speedup vs baseline: 1.1311x; 1.1311x over previous
import jax
import jax.numpy as jnp
from jax.experimental import pallas as pl

N_TOKENS = 9216
QUANT_DIM = 1024
OUTPUT_DIM = 256

_TB = 2304


def _body(x_ref, w_ref, out_ref):
    xb = x_ref[...]
    m = jnp.max(xb, axis=-1, keepdims=True)
    iota = jax.lax.broadcasted_iota(jnp.int32, (_TB, QUANT_DIM), 1)
    idx = jnp.min(jnp.where(xb == m, iota, QUANT_DIM), axis=-1)
    oh = (iota == idx[:, None]).astype(jnp.bfloat16)
    dims = (((1,), (1,)), ((), ()))
    out_ref[...] = jax.lax.dot_general(
        oh, w_ref[...].astype(jnp.bfloat16), dims,
        preferred_element_type=jnp.float32,
    )


def kernel(x, W):
    grid = N_TOKENS // _TB
    return pl.pallas_call(
        _body,
        grid=(grid,),
        in_specs=[
            pl.BlockSpec((_TB, QUANT_DIM), lambda i: (i, 0)),
            pl.BlockSpec((OUTPUT_DIM, QUANT_DIM), lambda i: (0, 0)),
        ],
        out_specs=pl.BlockSpec((_TB, OUTPUT_DIM), lambda i: (i, 0)),
        out_shape=jax.ShapeDtypeStruct((N_TOKENS, OUTPUT_DIM), jnp.float32),
    )(x, W)
